# baseline (speedup 1.0000x reference)
import functools

import jax
import jax.numpy as jnp
from jax import lax
from jax.experimental import pallas as pl
from jax.experimental.pallas import tpu as pltpu
from jax.experimental.pallas import tpu_sc as plsc

INPUT_WIDTH = 8192
OUTPUT_WIDTH = 1024
POPCNT_WIDTH = 64
BATCH = 1024

NUM_WORKERS = 32
ROWS_PER_WORKER = OUTPUT_WIDTH // NUM_WORKERS
LANES = 16
WORDS_PER_ROW = INPUT_WIDTH // 2


def _resilu(x):
    sig = 1.0 / (1.0 + jnp.exp(-x))
    return jnp.maximum(2.0 * sig - 1.0, 0.0)


def _bf16_bits(v):
    u = plsc.bitcast(v, jnp.int32)
    return (u + 0x7FFF + ((u >> 16) & 1)) >> 16


def _sc_build_w(sel_hbm, w_hbm, out_hbm, sel_v, wv_v, row_v, wrow_v):
    wid = lax.axis_index("s") * 2 + lax.axis_index("c")
    base = wid * ROWS_PER_WORKER

    pltpu.sync_copy(sel_hbm.at[pl.ds(base, ROWS_PER_WORKER)], sel_v)
    pltpu.sync_copy(w_hbm.at[pl.ds(base, ROWS_PER_WORKER)], wv_v)

    zeros16 = jnp.zeros((LANES,), jnp.float32)
    izeros16 = jnp.zeros((LANES,), jnp.int32)

    def _zero_f(i, carry):
        b0 = i * 128
        for j in range(8):
            row_v[pl.ds(b0 + j * LANES, LANES)] = zeros16
        return carry

    lax.fori_loop(0, INPUT_WIDTH // 128, _zero_f, 0)

    def _zero_i(i, carry):
        b0 = i * 128
        for j in range(8):
            wrow_v[pl.ds(b0 + j * LANES, LANES)] = izeros16
        return carry

    lax.fori_loop(0, WORDS_PER_ROW // 128, _zero_i, 0)

    lane = lax.iota(jnp.int32, LANES)

    def _row_body(r, carry):
        for j in range(POPCNT_WIDTH // LANES):
            idx = sel_v[r, pl.ds(j * LANES, LANES)]
            val = _resilu(wv_v[r, pl.ds(j * LANES, LANES)])
            for i in range(LANES):
                plsc.addupdate_scatter(row_v, [idx], val, mask=lane == i)
        for j in range(POPCNT_WIDTH // LANES):
            idx = sel_v[r, pl.ds(j * LANES, LANES)]
            g_even = plsc.load_gather(row_v, [idx & -2])
            g_odd = plsc.load_gather(row_v, [idx | 1])
            word = _bf16_bits(g_even) | (_bf16_bits(g_odd) << 16)
            plsc.store_scatter(wrow_v, [idx >> 1], word)
        pltpu.sync_copy(wrow_v, out_hbm.at[base + r])
        for j in range(POPCNT_WIDTH // LANES):
            idx = sel_v[r, pl.ds(j * LANES, LANES)]
            plsc.store_scatter(row_v, [idx], zeros16)
            plsc.store_scatter(wrow_v, [idx >> 1], izeros16)
        return carry

    lax.fori_loop(0, ROWS_PER_WORKER, _row_body, 0)


def _build_w(input_selection, weights):
    mesh = plsc.VectorSubcoreMesh(
        core_axis_name="c", subcore_axis_name="s", num_cores=2, num_subcores=16
    )
    return pl.kernel(
        _sc_build_w,
        out_type=jax.ShapeDtypeStruct((OUTPUT_WIDTH, WORDS_PER_ROW), jnp.int32),
        mesh=mesh,
        scratch_types=[
            pltpu.VMEM((ROWS_PER_WORKER, POPCNT_WIDTH), jnp.int32),
            pltpu.VMEM((ROWS_PER_WORKER, POPCNT_WIDTH), jnp.float32),
            pltpu.VMEM((INPUT_WIDTH,), jnp.float32),
            pltpu.VMEM((WORDS_PER_ROW,), jnp.int32),
        ],
        compiler_params=pltpu.CompilerParams(needs_layout_passes=False),
    )(input_selection, weights)


K_BLK = 2048


def _mm_kernel(x_ref, w_ref, b_ref, out_ref):
    k = pl.program_id(0)

    @pl.when(k == 0)
    def _():
        out_ref[...] = jnp.zeros_like(out_ref)

    out_ref[...] += lax.dot_general(
        x_ref[...].astype(jnp.bfloat16),
        w_ref[...],
        (((1,), (1,)), ((), ())),
        preferred_element_type=jnp.float32,
    )

    @pl.when(k == pl.num_programs(0) - 1)
    def _():
        out_ref[...] = _resilu(out_ref[...] - b_ref[...])


def _matmul(x, w_bf16, biases):
    grid = (INPUT_WIDTH // K_BLK,)
    return pl.pallas_call(
        _mm_kernel,
        grid=grid,
        in_specs=[
            pl.BlockSpec((BATCH, K_BLK), lambda k: (0, k)),
            pl.BlockSpec((OUTPUT_WIDTH, K_BLK), lambda k: (0, k)),
            pl.BlockSpec((1, OUTPUT_WIDTH), lambda k: (0, 0)),
        ],
        out_specs=pl.BlockSpec((BATCH, OUTPUT_WIDTH), lambda k: (0, 0)),
        out_shape=jax.ShapeDtypeStruct((BATCH, OUTPUT_WIDTH), jnp.float32),
    )(x, w_bf16, biases.reshape(1, OUTPUT_WIDTH))


def kernel(x, input_selection, weights, biases):
    w_words = _build_w(input_selection, weights)
    w_bf16 = lax.bitcast_convert_type(w_words, jnp.bfloat16).reshape(
        OUTPUT_WIDTH, INPUT_WIDTH
    )
    return _matmul(x, w_bf16, biases)


# speedup vs baseline: 2.9771x; 2.9771x over previous
import functools

import jax
import jax.numpy as jnp
from jax import lax
from jax.experimental import pallas as pl
from jax.experimental.pallas import tpu as pltpu
from jax.experimental.pallas import tpu_sc as plsc

INPUT_WIDTH = 8192
OUTPUT_WIDTH = 1024
POPCNT_WIDTH = 64
BATCH = 1024

NUM_WORKERS = 32
ROWS_PER_WORKER = OUTPUT_WIDTH // NUM_WORKERS
LANES = 16
WORDS_PER_ROW = INPUT_WIDTH // 2


def _resilu(x):
    sig = 1.0 / (1.0 + jnp.exp(-x))
    return jnp.maximum(2.0 * sig - 1.0, 0.0)


def _bf16_bits(v):
    u = plsc.bitcast(v, jnp.int32)
    return (u + 0x7FFF + ((u >> 16) & 1)) >> 16


def _sc_build_w(sel_hbm, w_hbm, out_hbm, sel_v, wv_v, row_v, wrow_v):
    wid = lax.axis_index("s") * 2 + lax.axis_index("c")
    base = wid * ROWS_PER_WORKER

    pltpu.sync_copy(sel_hbm.at[pl.ds(base, ROWS_PER_WORKER)], sel_v)
    pltpu.sync_copy(w_hbm.at[pl.ds(base, ROWS_PER_WORKER)], wv_v)

    zeros16 = jnp.zeros((LANES,), jnp.float32)
    izeros16 = jnp.zeros((LANES,), jnp.int32)

    def _zero_f(i, carry):
        b0 = i * 128
        for j in range(8):
            row_v[pl.ds(b0 + j * LANES, LANES)] = zeros16
        return carry

    lax.fori_loop(0, INPUT_WIDTH // 128, _zero_f, 0)

    def _zero_i(i, carry):
        b0 = i * 128
        for j in range(8):
            wrow_v[pl.ds(b0 + j * LANES, LANES)] = izeros16
        return carry

    lax.fori_loop(0, WORDS_PER_ROW // 128, _zero_i, 0)

    lane = lax.iota(jnp.int32, LANES)

    def _row_body(r, carry):
        for j in range(POPCNT_WIDTH // LANES):
            idx = sel_v[r, pl.ds(j * LANES, LANES)]
            val = _resilu(wv_v[r, pl.ds(j * LANES, LANES)])
            for i in range(LANES):
                plsc.addupdate_scatter(row_v, [idx], val, mask=lane == i)
        for j in range(POPCNT_WIDTH // LANES):
            idx = sel_v[r, pl.ds(j * LANES, LANES)]
            wa = idx & (WORDS_PER_ROW - 1)
            g_lo = plsc.load_gather(row_v, [wa])
            g_hi = plsc.load_gather(row_v, [wa | WORDS_PER_ROW])
            word = _bf16_bits(g_lo) | (_bf16_bits(g_hi) << 16)
            plsc.store_scatter(wrow_v, [wa], word)
        pltpu.sync_copy(wrow_v, out_hbm.at[base + r])
        for j in range(POPCNT_WIDTH // LANES):
            idx = sel_v[r, pl.ds(j * LANES, LANES)]
            plsc.store_scatter(row_v, [idx], zeros16)
            plsc.store_scatter(wrow_v, [idx & (WORDS_PER_ROW - 1)], izeros16)
        return carry

    lax.fori_loop(0, ROWS_PER_WORKER, _row_body, 0)


def _build_w(input_selection, weights):
    mesh = plsc.VectorSubcoreMesh(
        core_axis_name="c", subcore_axis_name="s", num_cores=2, num_subcores=16
    )
    return pl.kernel(
        _sc_build_w,
        out_type=jax.ShapeDtypeStruct((OUTPUT_WIDTH, WORDS_PER_ROW), jnp.int32),
        mesh=mesh,
        scratch_types=[
            pltpu.VMEM((ROWS_PER_WORKER, POPCNT_WIDTH), jnp.int32),
            pltpu.VMEM((ROWS_PER_WORKER, POPCNT_WIDTH), jnp.float32),
            pltpu.VMEM((INPUT_WIDTH,), jnp.float32),
            pltpu.VMEM((WORDS_PER_ROW,), jnp.int32),
        ],
        compiler_params=pltpu.CompilerParams(needs_layout_passes=False),
    )(input_selection, weights)


K_BLK = 2048


def _mm_kernel(x_ref, w_ref, b_ref, out_ref):
    g = pl.program_id(0)

    @pl.when(g == 0)
    def _():
        out_ref[...] = jnp.zeros_like(out_ref)

    wi = w_ref[...]
    shift = jnp.where((g & 1) == 0, 16, 0)
    wf = pltpu.bitcast((wi << shift) & jnp.int32(-65536), jnp.float32)
    out_ref[...] += lax.dot_general(
        x_ref[...].astype(jnp.bfloat16),
        wf.astype(jnp.bfloat16),
        (((1,), (1,)), ((), ())),
        preferred_element_type=jnp.float32,
    )

    @pl.when(g == pl.num_programs(0) - 1)
    def _():
        out_ref[...] = _resilu(out_ref[...] - b_ref[...])


def _matmul(x, w_words, biases):
    grid = (INPUT_WIDTH // K_BLK,)
    return pl.pallas_call(
        _mm_kernel,
        grid=grid,
        in_specs=[
            pl.BlockSpec((BATCH, K_BLK), lambda g: (0, (g & 1) * 2 + (g >> 1))),
            pl.BlockSpec((OUTPUT_WIDTH, K_BLK), lambda g: (0, g >> 1)),
            pl.BlockSpec((1, OUTPUT_WIDTH), lambda g: (0, 0)),
        ],
        out_specs=pl.BlockSpec((BATCH, OUTPUT_WIDTH), lambda g: (0, 0)),
        out_shape=jax.ShapeDtypeStruct((BATCH, OUTPUT_WIDTH), jnp.float32),
    )(x, w_words, biases.reshape(1, OUTPUT_WIDTH))


def kernel(x, input_selection, weights, biases):
    w_words = _build_w(input_selection, weights)
    return _matmul(x, w_words, biases)


# speedup vs baseline: 3.2403x; 1.0884x over previous
import functools

import jax
import jax.numpy as jnp
from jax import lax
from jax.experimental import pallas as pl
from jax.experimental.pallas import tpu as pltpu
from jax.experimental.pallas import tpu_sc as plsc

INPUT_WIDTH = 8192
OUTPUT_WIDTH = 1024
POPCNT_WIDTH = 64
BATCH = 1024

NUM_WORKERS = 32
ROWS_PER_WORKER = OUTPUT_WIDTH // NUM_WORKERS
LANES = 16
WORDS_PER_ROW = INPUT_WIDTH // 2


def _resilu(x):
    sig = 1.0 / (1.0 + jnp.exp(-x))
    return jnp.maximum(2.0 * sig - 1.0, 0.0)


def _bf16_bits(v):
    u = plsc.bitcast(v, jnp.int32)
    return (u + 0x7FFF + ((u >> 16) & 1)) >> 16


def _sc_build_w(sel_hbm, w_hbm, out_hbm, sel_v, wv_v, row_v, wrow_a, wrow_b, sem_a, sem_b):
    wid = lax.axis_index("s") * 2 + lax.axis_index("c")
    base = wid * ROWS_PER_WORKER

    pltpu.sync_copy(sel_hbm.at[pl.ds(base, ROWS_PER_WORKER)], sel_v)
    pltpu.sync_copy(w_hbm.at[pl.ds(base, ROWS_PER_WORKER)], wv_v)

    zeros16 = jnp.zeros((LANES,), jnp.float32)
    izeros16 = jnp.zeros((LANES,), jnp.int32)

    def _zero_f(i, carry):
        b0 = i * 128
        for j in range(8):
            row_v[pl.ds(b0 + j * LANES, LANES)] = zeros16
        return carry

    lax.fori_loop(0, INPUT_WIDTH // 128, _zero_f, 0)

    def _zero_i(i, carry):
        b0 = i * 128
        for wrow in (wrow_a, wrow_b):
            for j in range(8):
                wrow[pl.ds(b0 + j * LANES, LANES)] = izeros16
        return carry

    lax.fori_loop(0, WORDS_PER_ROW // 128, _zero_i, 0)

    lane = lax.iota(jnp.int32, LANES)
    bufs = ((wrow_a, sem_a), (wrow_b, sem_b))

    def _build_row(r, wrow):
        for j in range(POPCNT_WIDTH // LANES):
            idx = sel_v[r, pl.ds(j * LANES, LANES)]
            val = _resilu(wv_v[r, pl.ds(j * LANES, LANES)])
            for i in range(LANES):
                plsc.addupdate_scatter(row_v, [idx], val, mask=lane == i)
        for j in range(POPCNT_WIDTH // LANES):
            idx = sel_v[r, pl.ds(j * LANES, LANES)]
            wa = idx & (WORDS_PER_ROW - 1)
            g_lo = plsc.load_gather(row_v, [wa])
            g_hi = plsc.load_gather(row_v, [wa | WORDS_PER_ROW])
            word = _bf16_bits(g_lo) | (_bf16_bits(g_hi) << 16)
            plsc.store_scatter(wrow, [wa], word)
        for j in range(POPCNT_WIDTH // LANES):
            idx = sel_v[r, pl.ds(j * LANES, LANES)]
            plsc.store_scatter(row_v, [idx], zeros16)

    def _pair_body(p, carry):
        for b, (wrow, sem) in enumerate(bufs):
            r = 2 * p + b

            @pl.when(p >= 1)
            def _(wrow=wrow, sem=sem, r=r):
                pltpu.make_async_copy(
                    wrow, out_hbm.at[base + r - 2], sem
                ).wait()
                for j in range(POPCNT_WIDTH // LANES):
                    old = sel_v[r - 2, pl.ds(j * LANES, LANES)]
                    plsc.store_scatter(
                        wrow, [old & (WORDS_PER_ROW - 1)], izeros16
                    )

            _build_row(r, wrow)
            pltpu.async_copy(wrow, out_hbm.at[base + r], sem)
        return carry

    lax.fori_loop(0, ROWS_PER_WORKER // 2, _pair_body, 0)

    for b, (wrow, sem) in enumerate(bufs):
        r_last = ROWS_PER_WORKER - 2 + b
        pltpu.make_async_copy(wrow, out_hbm.at[base + r_last], sem).wait()


def _build_w(input_selection, weights):
    mesh = plsc.VectorSubcoreMesh(
        core_axis_name="c", subcore_axis_name="s", num_cores=2, num_subcores=16
    )
    return pl.kernel(
        _sc_build_w,
        out_type=jax.ShapeDtypeStruct((OUTPUT_WIDTH, WORDS_PER_ROW), jnp.int32),
        mesh=mesh,
        scratch_types=[
            pltpu.VMEM((ROWS_PER_WORKER, POPCNT_WIDTH), jnp.int32),
            pltpu.VMEM((ROWS_PER_WORKER, POPCNT_WIDTH), jnp.float32),
            pltpu.VMEM((INPUT_WIDTH,), jnp.float32),
            pltpu.VMEM((WORDS_PER_ROW,), jnp.int32),
            pltpu.VMEM((WORDS_PER_ROW,), jnp.int32),
            pltpu.SemaphoreType.DMA,
            pltpu.SemaphoreType.DMA,
        ],
        compiler_params=pltpu.CompilerParams(needs_layout_passes=False),
    )(input_selection, weights)


K_BLK = 2048


def _mm_kernel(x_ref, w_ref, b_ref, out_ref):
    g = pl.program_id(0)

    @pl.when(g == 0)
    def _():
        out_ref[...] = jnp.zeros_like(out_ref)

    wi = w_ref[...]
    shift = jnp.where((g & 1) == 0, 16, 0)
    wf = pltpu.bitcast((wi << shift) & jnp.int32(-65536), jnp.float32)
    out_ref[...] += lax.dot_general(
        x_ref[...].astype(jnp.bfloat16),
        wf.astype(jnp.bfloat16),
        (((1,), (1,)), ((), ())),
        preferred_element_type=jnp.float32,
    )

    @pl.when(g == pl.num_programs(0) - 1)
    def _():
        out_ref[...] = _resilu(out_ref[...] - b_ref[...])


def _matmul(x, w_words, biases):
    grid = (INPUT_WIDTH // K_BLK,)
    return pl.pallas_call(
        _mm_kernel,
        grid=grid,
        in_specs=[
            pl.BlockSpec((BATCH, K_BLK), lambda g: (0, (g & 1) * 2 + (g >> 1))),
            pl.BlockSpec((OUTPUT_WIDTH, K_BLK), lambda g: (0, g >> 1)),
            pl.BlockSpec((1, OUTPUT_WIDTH), lambda g: (0, 0)),
        ],
        out_specs=pl.BlockSpec((BATCH, OUTPUT_WIDTH), lambda g: (0, 0)),
        out_shape=jax.ShapeDtypeStruct((BATCH, OUTPUT_WIDTH), jnp.float32),
    )(x, w_words, biases.reshape(1, OUTPUT_WIDTH))


def kernel(x, input_selection, weights, biases):
    w_words = _build_w(input_selection, weights)
    return _matmul(x, w_words, biases)
